# Initial kernel scaffold; baseline (speedup 1.0000x reference)
import jax
import jax.numpy as jnp
from jax.experimental import pallas as pl


def kernel(x, enc_W0, enc_b0, enc_W1, enc_b1, dec_W0, dec_b0, dec_W1, dec_b1, codebooks):
    raise NotImplementedError("write your pallas kernel here")


# speedup vs baseline: 1.2922x; 1.2922x over previous
import functools

import jax
import jax.numpy as jnp
from jax import lax
from jax.experimental import pallas as pl
from jax.experimental.pallas import tpu as pltpu
from jax.experimental.pallas import tpu_sc as plsc

NUM_Q = 8
K = 8192
D = 256
T = 8192
BT_VQ = 256
BT_MLP = 512

SC_CORES = 2
SC_SUBCORES = 16
NW = SC_CORES * SC_SUBCORES
TOK_PER_W = T // NW
GCH = 128
CH_PER_W = TOK_PER_W // GCH


def _enc_body(x_ref, w0_ref, b0_ref, w1_ref, b1_ref, z_ref):
    h = jnp.dot(x_ref[...], w0_ref[...], preferred_element_type=jnp.float32)
    h = jnp.maximum(h + b0_ref[...], 0.0)
    z_ref[...] = jnp.dot(h, w1_ref[...], preferred_element_type=jnp.float32) + b1_ref[...]


def _encoder(xf, w0, b0, w1, b1):
    nb = T // BT_MLP
    return pl.pallas_call(
        _enc_body,
        grid=(nb,),
        in_specs=[
            pl.BlockSpec((BT_MLP, 768), lambda i: (i, 0)),
            pl.BlockSpec((768, 512), lambda i: (0, 0)),
            pl.BlockSpec((1, 512), lambda i: (0, 0)),
            pl.BlockSpec((512, 256), lambda i: (0, 0)),
            pl.BlockSpec((1, 256), lambda i: (0, 0)),
        ],
        out_specs=pl.BlockSpec((BT_MLP, 256), lambda i: (i, 0)),
        out_shape=jax.ShapeDtypeStruct((T, 256), jnp.float32),
    )(xf, w0, b0, w1, b1)


def _vq_body_first(r_ref, cb_ref, cc_ref, idx_ref):
    r = r_ref[...]
    rc = lax.dot_general(r, cb_ref[...], (((1,), (1,)), ((), ())),
                         preferred_element_type=jnp.float32)
    rr = jnp.sum(r * r, axis=1, keepdims=True)
    d = (rr - 2.0 * rc) + cc_ref[...]
    m = jnp.min(d, axis=1, keepdims=True)
    ii = lax.broadcasted_iota(jnp.int32, d.shape, 1)
    idx_ref[0, 0, :] = jnp.min(jnp.where(d == m, ii, K), axis=1)


def _vq_body(rp_ref, qp_ref, cb_ref, cc_ref, idx_ref, r_ref):
    r = rp_ref[...] - qp_ref[...]
    r_ref[...] = r
    rc = lax.dot_general(r, cb_ref[...], (((1,), (1,)), ((), ())),
                         preferred_element_type=jnp.float32)
    rr = jnp.sum(r * r, axis=1, keepdims=True)
    d = (rr - 2.0 * rc) + cc_ref[...]
    m = jnp.min(d, axis=1, keepdims=True)
    ii = lax.broadcasted_iota(jnp.int32, d.shape, 1)
    idx_ref[0, 0, :] = jnp.min(jnp.where(d == m, ii, K), axis=1)


def _vq_argmin_first(r, cb, cc):
    nb = T // BT_VQ
    return pl.pallas_call(
        _vq_body_first,
        grid=(nb,),
        in_specs=[
            pl.BlockSpec((BT_VQ, D), lambda i: (i, 0)),
            pl.BlockSpec((K, D), lambda i: (0, 0)),
            pl.BlockSpec((1, K), lambda i: (0, 0)),
        ],
        out_specs=pl.BlockSpec((1, 1, BT_VQ), lambda i: (i, 0, 0)),
        out_shape=jax.ShapeDtypeStruct((nb, 1, BT_VQ), jnp.int32),
    )(r, cb, cc)


def _vq_argmin(r_prev, q_prev, cb, cc):
    nb = T // BT_VQ
    return pl.pallas_call(
        _vq_body,
        grid=(nb,),
        in_specs=[
            pl.BlockSpec((BT_VQ, D), lambda i: (i, 0)),
            pl.BlockSpec((BT_VQ, D), lambda i: (i, 0)),
            pl.BlockSpec((K, D), lambda i: (0, 0)),
            pl.BlockSpec((1, K), lambda i: (0, 0)),
        ],
        out_specs=[
            pl.BlockSpec((1, 1, BT_VQ), lambda i: (i, 0, 0)),
            pl.BlockSpec((BT_VQ, D), lambda i: (i, 0)),
        ],
        out_shape=[
            jax.ShapeDtypeStruct((nb, 1, BT_VQ), jnp.int32),
            jax.ShapeDtypeStruct((T, D), jnp.float32),
        ],
    )(r_prev, q_prev, cb, cc)


def _sc_gather_body(cb_hbm, idx_hbm, out_hbm, idx_v, rows_v, sem):
    wid = lax.axis_index("c") * SC_SUBCORES + lax.axis_index("s")
    pltpu.sync_copy(idx_hbm.at[pl.ds(wid * CH_PER_W, CH_PER_W)], idx_v)
    for j in range(CH_PER_W):
        pltpu.async_copy(cb_hbm.at[idx_v.at[j]], rows_v, sem).wait()
        pltpu.sync_copy(rows_v,
                        out_hbm.at[pl.ds(wid * TOK_PER_W + j * GCH, GCH)])


@functools.lru_cache(maxsize=1)
def _sc_gather_call():
    return functools.partial(
        pl.kernel,
        mesh=plsc.VectorSubcoreMesh(core_axis_name="c", subcore_axis_name="s",
                                    num_cores=SC_CORES),
        out_type=jax.ShapeDtypeStruct((T, D), jnp.float32),
        scratch_types=[
            pltpu.VMEM((CH_PER_W, GCH), jnp.int32),
            pltpu.VMEM((GCH, D), jnp.float32),
            pltpu.SemaphoreType.DMA,
        ],
    )(_sc_gather_body)


def _sc_gather(cb, idx2):
    return _sc_gather_call()(cb, idx2)


def _dec_body(z_ref, r_ref, q_ref, w0_ref, b0_ref, w1_ref, b1_ref, out_ref):
    q = z_ref[...] - (r_ref[...] - q_ref[...])
    h = jnp.dot(q, w0_ref[...], preferred_element_type=jnp.float32)
    h = jnp.maximum(h + b0_ref[...], 0.0)
    out_ref[...] = jnp.dot(h, w1_ref[...], preferred_element_type=jnp.float32) + b1_ref[...]


def _decoder(z, r_last, q_last, w0, b0, w1, b1):
    nb = T // BT_MLP
    return pl.pallas_call(
        _dec_body,
        grid=(nb,),
        in_specs=[
            pl.BlockSpec((BT_MLP, 256), lambda i: (i, 0)),
            pl.BlockSpec((BT_MLP, 256), lambda i: (i, 0)),
            pl.BlockSpec((BT_MLP, 256), lambda i: (i, 0)),
            pl.BlockSpec((256, 512), lambda i: (0, 0)),
            pl.BlockSpec((1, 512), lambda i: (0, 0)),
            pl.BlockSpec((512, 768), lambda i: (0, 0)),
            pl.BlockSpec((1, 768), lambda i: (0, 0)),
        ],
        out_specs=pl.BlockSpec((BT_MLP, 768), lambda i: (i, 0)),
        out_shape=jax.ShapeDtypeStruct((T, 768), jnp.float32),
    )(z, r_last, q_last, w0, b0, w1, b1)


def kernel(x, enc_W0, enc_b0, enc_W1, enc_b1, dec_W0, dec_b0, dec_W1, dec_b1, codebooks):
    B, N, F = x.shape
    xf = x.reshape(T, F)
    z = _encoder(xf, enc_W0, enc_b0.reshape(1, -1), enc_W1, enc_b1.reshape(1, -1))

    r = z
    q = None
    for i in range(NUM_Q):
        cb = codebooks[i]
        cc = jnp.sum(cb ** 2, axis=-1).reshape(1, K)
        if i == 0:
            idx = _vq_argmin_first(r, cb, cc)
        else:
            idx, r = _vq_argmin(r, q, cb, cc)
        q = _sc_gather(cb, idx.reshape(T // GCH, GCH))

    out = _decoder(z, r, q, dec_W0, dec_b0.reshape(1, -1),
                   dec_W1, dec_b1.reshape(1, -1))
    return out.reshape(B, N, 768)


# speedup vs baseline: 1.6665x; 1.2897x over previous
import functools

import jax
import jax.numpy as jnp
from jax import lax
from jax.experimental import pallas as pl
from jax.experimental.pallas import tpu as pltpu
from jax.experimental.pallas import tpu_sc as plsc

NUM_Q = 8
K = 8192
D = 256
T = 8192
TH = T // 2
BT_VQ = 512
NB_H = TH // BT_VQ
BT_MLP = 512

SC_CORES = 2
SC_SUBCORES = 16
NW = SC_CORES * SC_SUBCORES
GCH = TH // NW


def _enc_body(x_ref, w0_ref, b0_ref, w1_ref, b1_ref, z_ref):
    h = jnp.dot(x_ref[...], w0_ref[...], preferred_element_type=jnp.float32)
    h = jnp.maximum(h + b0_ref[...], 0.0)
    z_ref[...] = jnp.dot(h, w1_ref[...], preferred_element_type=jnp.float32) + b1_ref[...]


def _encoder(xf, w0, b0, w1, b1):
    nb = T // BT_MLP
    return pl.pallas_call(
        _enc_body,
        grid=(nb,),
        in_specs=[
            pl.BlockSpec((BT_MLP, 768), lambda i: (i, 0)),
            pl.BlockSpec((768, 512), lambda i: (0, 0)),
            pl.BlockSpec((1, 512), lambda i: (0, 0)),
            pl.BlockSpec((512, 256), lambda i: (0, 0)),
            pl.BlockSpec((1, 256), lambda i: (0, 0)),
        ],
        out_specs=pl.BlockSpec((BT_MLP, 256), lambda i: (i, 0)),
        out_shape=jax.ShapeDtypeStruct((T, 256), jnp.float32),
    )(xf, w0, b0, w1, b1)


def _vq_body_first(r_ref, cb_ref, cc_ref, idx_ref):
    r = r_ref[...]
    rc = lax.dot_general(r, cb_ref[...], (((1,), (1,)), ((), ())),
                         preferred_element_type=jnp.float32)
    rr = jnp.sum(r * r, axis=1, keepdims=True)
    d = (rr - 2.0 * rc) + cc_ref[...]
    idx_ref[0, 0, :] = jnp.argmin(d, axis=1).astype(jnp.int32)


def _vq_body(rp_ref, qp_ref, cb_ref, cc_ref, idx_ref, r_ref):
    r = rp_ref[...] - qp_ref[...]
    r_ref[...] = r
    rc = lax.dot_general(r, cb_ref[...], (((1,), (1,)), ((), ())),
                         preferred_element_type=jnp.float32)
    rr = jnp.sum(r * r, axis=1, keepdims=True)
    d = (rr - 2.0 * rc) + cc_ref[...]
    idx_ref[0, 0, :] = jnp.argmin(d, axis=1).astype(jnp.int32)


def _vq_argmin_first(z, cb, cc, half):
    return pl.pallas_call(
        _vq_body_first,
        grid=(NB_H,),
        in_specs=[
            pl.BlockSpec((BT_VQ, D), lambda i: (i + half * NB_H, 0)),
            pl.BlockSpec((K, D), lambda i: (0, 0)),
            pl.BlockSpec((1, K), lambda i: (0, 0)),
        ],
        out_specs=pl.BlockSpec((1, 1, BT_VQ), lambda i: (i, 0, 0)),
        out_shape=jax.ShapeDtypeStruct((NB_H, 1, BT_VQ), jnp.int32),
    )(z, cb, cc)


def _vq_argmin(r_prev, q_prev, cb, cc, off=0):
    return pl.pallas_call(
        _vq_body,
        grid=(NB_H,),
        in_specs=[
            pl.BlockSpec((BT_VQ, D), lambda i: (i + off, 0)),
            pl.BlockSpec((BT_VQ, D), lambda i: (i, 0)),
            pl.BlockSpec((K, D), lambda i: (0, 0)),
            pl.BlockSpec((1, K), lambda i: (0, 0)),
        ],
        out_specs=[
            pl.BlockSpec((1, 1, BT_VQ), lambda i: (i, 0, 0)),
            pl.BlockSpec((BT_VQ, D), lambda i: (i, 0)),
        ],
        out_shape=[
            jax.ShapeDtypeStruct((NB_H, 1, BT_VQ), jnp.int32),
            jax.ShapeDtypeStruct((TH, D), jnp.float32),
        ],
    )(r_prev, q_prev, cb, cc)


def _sc_gather_body(cb_hbm, idx_hbm, out_hbm, idx_v, rows_v, sem):
    wid = lax.axis_index("c") * SC_SUBCORES + lax.axis_index("s")
    pltpu.sync_copy(idx_hbm.at[pl.ds(wid, 1)], idx_v)
    pltpu.async_copy(cb_hbm.at[idx_v.at[0]], rows_v, sem).wait()
    pltpu.sync_copy(rows_v, out_hbm.at[pl.ds(wid * GCH, GCH)])


@functools.lru_cache(maxsize=1)
def _sc_gather_call():
    return functools.partial(
        pl.kernel,
        mesh=plsc.VectorSubcoreMesh(core_axis_name="c", subcore_axis_name="s",
                                    num_cores=SC_CORES),
        out_type=jax.ShapeDtypeStruct((TH, D), jnp.float32),
        scratch_types=[
            pltpu.VMEM((1, GCH), jnp.int32),
            pltpu.VMEM((GCH, D), jnp.float32),
            pltpu.SemaphoreType.DMA,
        ],
    )(_sc_gather_body)


def _sc_gather(cb, idx2):
    return _sc_gather_call()(cb, idx2)


def _dec_body(z_ref, ra_ref, rb_ref, qa_ref, qb_ref,
              w0_ref, b0_ref, w1_ref, b1_ref, out_ref):
    in_a = pl.program_id(0) < NB_H
    r = jnp.where(in_a, ra_ref[...], rb_ref[...])
    qp = jnp.where(in_a, qa_ref[...], qb_ref[...])
    q = z_ref[...] - (r - qp)
    h = jnp.dot(q, w0_ref[...], preferred_element_type=jnp.float32)
    h = jnp.maximum(h + b0_ref[...], 0.0)
    out_ref[...] = jnp.dot(h, w1_ref[...], preferred_element_type=jnp.float32) + b1_ref[...]


def _decoder(z, r_halves, q_halves, w0, b0, w1, b1):
    nb = T // BT_MLP
    half_spec = [
        pl.BlockSpec((BT_MLP, 256), lambda i: (jnp.minimum(i, NB_H - 1), 0)),
        pl.BlockSpec((BT_MLP, 256), lambda i: (jnp.maximum(i - NB_H, 0), 0)),
    ]
    return pl.pallas_call(
        _dec_body,
        grid=(nb,),
        in_specs=[
            pl.BlockSpec((BT_MLP, 256), lambda i: (i, 0)),
            *half_spec,
            *half_spec,
            pl.BlockSpec((256, 512), lambda i: (0, 0)),
            pl.BlockSpec((1, 512), lambda i: (0, 0)),
            pl.BlockSpec((512, 768), lambda i: (0, 0)),
            pl.BlockSpec((1, 768), lambda i: (0, 0)),
        ],
        out_specs=pl.BlockSpec((BT_MLP, 768), lambda i: (i, 0)),
        out_shape=jax.ShapeDtypeStruct((T, 768), jnp.float32),
    )(z, r_halves[0], r_halves[1], q_halves[0], q_halves[1], w0, b0, w1, b1)


def kernel(x, enc_W0, enc_b0, enc_W1, enc_b1, dec_W0, dec_b0, dec_W1, dec_b1, codebooks):
    B, N, F = x.shape
    xf = x.reshape(T, F)
    z = _encoder(xf, enc_W0, enc_b0.reshape(1, -1), enc_W1, enc_b1.reshape(1, -1))

    r = [None, None]
    q = [None, None]
    for i in range(NUM_Q):
        cb = codebooks[i]
        cc = jnp.sum(cb ** 2, axis=-1).reshape(1, K)
        for h in range(2):
            if i == 0:
                idx = _vq_argmin_first(z, cb, cc, h)
            elif i == 1:
                idx, r[h] = _vq_argmin(z, q[h], cb, cc, off=h * NB_H)
            else:
                idx, r[h] = _vq_argmin(r[h], q[h], cb, cc)
            q[h] = _sc_gather(cb, idx.reshape(NW, GCH))

    out = _decoder(z, r, q, dec_W0, dec_b0.reshape(1, -1),
                   dec_W1, dec_b1.reshape(1, -1))
    return out.reshape(B, N, 768)


# speedup vs baseline: 1.8135x; 1.0882x over previous
import functools

import jax
import jax.numpy as jnp
from jax import lax
from jax.experimental import pallas as pl
from jax.experimental.pallas import tpu as pltpu
from jax.experimental.pallas import tpu_sc as plsc

NUM_Q = 8
K = 8192
D = 256
T = 8192
TH = T // 2
BT_VQ = 512
NB_H = TH // BT_VQ
BT_MLP = 512

SC_CORES = 2
SC_SUBCORES = 16
NW = SC_CORES * SC_SUBCORES
GCH = TH // NW


def _enc_body(x_ref, w0_ref, b0_ref, w1_ref, b1_ref, z_ref):
    h = jnp.dot(x_ref[...], w0_ref[...], preferred_element_type=jnp.float32)
    h = jnp.maximum(h + b0_ref[...], 0.0)
    z_ref[...] = jnp.dot(h, w1_ref[...], preferred_element_type=jnp.float32) + b1_ref[...]


def _encoder(xf, w0, b0, w1, b1):
    nb = T // BT_MLP
    return pl.pallas_call(
        _enc_body,
        grid=(nb,),
        in_specs=[
            pl.BlockSpec((BT_MLP, 768), lambda i: (i, 0)),
            pl.BlockSpec((768, 512), lambda i: (0, 0)),
            pl.BlockSpec((1, 512), lambda i: (0, 0)),
            pl.BlockSpec((512, 256), lambda i: (0, 0)),
            pl.BlockSpec((1, 256), lambda i: (0, 0)),
        ],
        out_specs=pl.BlockSpec((BT_MLP, 256), lambda i: (i, 0)),
        out_shape=jax.ShapeDtypeStruct((T, 256), jnp.float32),
    )(xf, w0, b0, w1, b1)


def _make_vq_body(idx_off):
    def body(r_ref, cb_ref, cc_ref, idx_ref):
        r = r_ref[...]
        rc = lax.dot_general(r, cb_ref[0], (((1,), (1,)), ((), ())),
                             preferred_element_type=jnp.float32)
        rr = jnp.sum(r * r, axis=1, keepdims=True)
        d = (rr - 2.0 * rc) + cc_ref[0]
        idx_ref[0, 0, :] = jnp.argmin(d, axis=1).astype(jnp.int32) + idx_off
    return body


@functools.lru_cache(maxsize=None)
def _vq_argmin_call(layer, roff):
    return pl.pallas_call(
        _make_vq_body(layer * K),
        grid=(NB_H,),
        in_specs=[
            pl.BlockSpec((BT_VQ, D), lambda i: (i + roff, 0)),
            pl.BlockSpec((1, K, D), lambda i: (layer, 0, 0)),
            pl.BlockSpec((1, 1, K), lambda i: (layer, 0, 0)),
        ],
        out_specs=pl.BlockSpec((1, 1, BT_VQ), lambda i: (i, 0, 0)),
        out_shape=jax.ShapeDtypeStruct((NB_H, 1, BT_VQ), jnp.int32),
    )


def _make_sc_body(roff):
    def body(cb_hbm, idx_hbm, rp_hbm, out_hbm, idx_v, rows_v, rp_v, sem):
        wid = lax.axis_index("c") * SC_SUBCORES + lax.axis_index("s")
        pltpu.sync_copy(idx_hbm.at[pl.ds(wid, 1)], idx_v)
        gather = pltpu.async_copy(cb_hbm.at[idx_v.at[0]], rows_v, sem)
        pltpu.sync_copy(rp_hbm.at[pl.ds(roff + wid * GCH, GCH)], rp_v)
        gather.wait()

        def row_fn(i, carry):
            for c in range(D // 16):
                sl = pl.ds(c * 16, 16)
                rp_v[i, sl] = rp_v[i, sl] - rows_v[i, sl]
            return carry

        lax.fori_loop(0, GCH, row_fn, 0)
        pltpu.sync_copy(rp_v, out_hbm.at[pl.ds(wid * GCH, GCH)])
    return body


@functools.lru_cache(maxsize=None)
def _sc_update_call(rp_rows, roff):
    return functools.partial(
        pl.kernel,
        mesh=plsc.VectorSubcoreMesh(core_axis_name="c", subcore_axis_name="s",
                                    num_cores=SC_CORES),
        out_type=jax.ShapeDtypeStruct((TH, D), jnp.float32),
        scratch_types=[
            pltpu.VMEM((1, GCH), jnp.int32),
            pltpu.VMEM((GCH, D), jnp.float32),
            pltpu.VMEM((GCH, D), jnp.float32),
            pltpu.SemaphoreType.DMA,
        ],
    )(_make_sc_body(roff))


def _sc_update(cb_flat, idx2, r_prev, roff):
    return _sc_update_call(r_prev.shape[0], roff)(cb_flat, idx2, r_prev)


def _dec_body(z_ref, ra_ref, rb_ref, w0_ref, b0_ref, w1_ref, b1_ref, out_ref):
    r = jnp.where(pl.program_id(0) < NB_H, ra_ref[...], rb_ref[...])
    q = z_ref[...] - r
    h = jnp.dot(q, w0_ref[...], preferred_element_type=jnp.float32)
    h = jnp.maximum(h + b0_ref[...], 0.0)
    out_ref[...] = jnp.dot(h, w1_ref[...], preferred_element_type=jnp.float32) + b1_ref[...]


def _decoder(z, ra, rb, w0, b0, w1, b1):
    nb = T // BT_MLP
    return pl.pallas_call(
        _dec_body,
        grid=(nb,),
        in_specs=[
            pl.BlockSpec((BT_MLP, 256), lambda i: (i, 0)),
            pl.BlockSpec((BT_MLP, 256), lambda i: (jnp.minimum(i, NB_H - 1), 0)),
            pl.BlockSpec((BT_MLP, 256), lambda i: (jnp.maximum(i - NB_H, 0), 0)),
            pl.BlockSpec((256, 512), lambda i: (0, 0)),
            pl.BlockSpec((1, 512), lambda i: (0, 0)),
            pl.BlockSpec((512, 768), lambda i: (0, 0)),
            pl.BlockSpec((1, 768), lambda i: (0, 0)),
        ],
        out_specs=pl.BlockSpec((BT_MLP, 768), lambda i: (i, 0)),
        out_shape=jax.ShapeDtypeStruct((T, 768), jnp.float32),
    )(z, ra, rb, w0, b0, w1, b1)


def kernel(x, enc_W0, enc_b0, enc_W1, enc_b1, dec_W0, dec_b0, dec_W1, dec_b1, codebooks):
    B, N, F = x.shape
    xf = x.reshape(T, F)
    z = _encoder(xf, enc_W0, enc_b0.reshape(1, -1), enc_W1, enc_b1.reshape(1, -1))

    cc_all = jnp.sum(codebooks ** 2, axis=-1).reshape(NUM_Q, 1, K)
    cb_flat = codebooks.reshape(NUM_Q * K, D)

    r = [None, None]
    for i in range(NUM_Q):
        for h in range(2):
            if i == 0:
                idx = _vq_argmin_call(i, h * NB_H)(z, codebooks, cc_all)
                r[h] = _sc_update(cb_flat, idx.reshape(NW, GCH), z, h * TH)
            else:
                idx = _vq_argmin_call(i, 0)(r[h], codebooks, cc_all)
                r[h] = _sc_update(cb_flat, idx.reshape(NW, GCH), r[h], 0)

    out = _decoder(z, r[0], r[1], dec_W0, dec_b0.reshape(1, -1),
                   dec_W1, dec_b1.reshape(1, -1))
    return out.reshape(B, N, 768)
